# baseline (speedup 1.0000x reference)
import functools

import jax
import jax.numpy as jnp
import numpy as np
from jax import lax
from jax.experimental import pallas as pl
from jax.experimental.pallas import tpu as pltpu
from jax.experimental.pallas import tpu_sc as plsc

BS, LS = 32, 64
N = BS * LS
WDIM, CDIM, PDIM = 128, 32, 48
MAXC = 20
INP = WDIM + CDIM + PDIM
CTX = 128
HS = 2 * CTX
GS = (2000, 1500, 1000, 800)
POS_IDS = (1, 2, 3, 4)
NW = 32


def _sc_gather_rows(table, idx):
    B = idx.shape[0]
    D = table.shape[1]
    bpw = B // NW
    mesh = plsc.VectorSubcoreMesh(core_axis_name="c", subcore_axis_name="s")

    @functools.partial(
        pl.kernel,
        out_type=jax.ShapeDtypeStruct((B, D), table.dtype),
        mesh=mesh,
        scratch_types=[
            pltpu.VMEM((bpw,), jnp.int32),
            pltpu.VMEM((bpw, D), table.dtype),
            pltpu.SemaphoreType.DMA,
        ],
    )
    def k(table_hbm, idx_hbm, out_hbm, idx_v, rows_v, sem):
        wid = lax.axis_index("s") * 2 + lax.axis_index("c")
        base = wid * bpw
        pltpu.sync_copy(idx_hbm.at[pl.ds(base, bpw)], idx_v)
        pltpu.async_copy(table_hbm.at[idx_v], rows_v, sem).wait()
        pltpu.sync_copy(rows_v, out_hbm.at[pl.ds(base, bpw)])

    return k(table, idx)


_PACK = 1 << 17


def _np_threefry(k1, k2, x0, x1):
    k1 = np.uint32(k1)
    k2 = np.uint32(k2)
    x0 = np.uint32(x0)
    x1 = np.uint32(x1)
    ks = (k1, k2, np.uint32(k1 ^ k2 ^ np.uint32(0x1BD11BDA)))
    r1 = (13, 15, 26, 6)
    r2 = (17, 29, 16, 24)

    def rol(v, d):
        return np.uint32((np.uint32(v << np.uint32(d)))
                         | np.uint32(v >> np.uint32(32 - d)))

    def rounds(a, b, rots):
        for r in rots:
            a = np.uint32(a + b)
            b = rol(b, r)
            b = np.uint32(a ^ b)
        return a, b

    x0 = np.uint32(x0 + ks[0])
    x1 = np.uint32(x1 + ks[1])
    for i, rots in enumerate((r1, r2, r1, r2, r1)):
        x0, x1 = rounds(x0, x1, rots)
        x0 = np.uint32(x0 + ks[(i + 1) % 3])
        x1 = np.uint32(x1 + ks[(i + 2) % 3] + np.uint32(i + 1))
    return x0, x1


with np.errstate(over="ignore"):
    _GKEYS = tuple(_np_threefry(0, 42, 0, gi) for gi in range(4))
_TINY = np.float32(np.finfo(np.float32).tiny)


def _gumbel_tile(gi, rowids, M):
    rows = rowids.shape[0]
    k1, k2 = _GKEYS[gi]
    ks0 = jnp.uint32(k1)
    ks1 = jnp.uint32(k2)
    ks2 = jnp.uint32(np.uint32(k1) ^ np.uint32(k2) ^ np.uint32(0x1BD11BDA))
    j = lax.broadcasted_iota(jnp.int32, (rows, M), 1)
    n = (rowids * M + j).astype(jnp.uint32)

    def rol(v, d):
        return (v << jnp.uint32(d)) | (v >> jnp.uint32(32 - d))

    def rounds(a, b, rots):
        for rr in rots:
            a = a + b
            b = rol(b, rr)
            b = a ^ b
        return a, b

    x0 = jnp.full((rows, M), ks0, jnp.uint32)
    x1 = n + ks1
    ksl = (ks0, ks1, ks2)
    r1 = (13, 15, 26, 6)
    r2 = (17, 29, 16, 24)
    for i, rots in enumerate((r1, r2, r1, r2, r1)):
        x0, x1 = rounds(x0, x1, rots)
        x0 = x0 + ksl[(i + 1) % 3]
        x1 = x1 + (ksl[(i + 2) % 3] + jnp.uint32(i + 1))
    bits = x0 ^ x1
    fb = (bits >> jnp.uint32(9)) | jnp.uint32(0x3F800000)
    f = lax.bitcast_convert_type(fb, jnp.float32) - 1.0
    u = jnp.maximum(_TINY, f * np.float32(1.0 - _TINY) + _TINY)
    return -jnp.log(-jnp.log(u))


def _lstm_kernel(wt_ref, char_ref, post_ref, mask_ref, cemb_ref, pemb_ref,
                 wih_ref, whh_ref, bias_ref, ctx_ref, x_ref, xw_ref):
    cnt = jnp.zeros((N, 128), jnp.float32)
    for j in range(MAXC):
        col = char_ref[:, j][:, None]
        cnt = cnt + (col == lax.broadcasted_iota(jnp.int32, (N, 128), 1)
                     ).astype(jnp.float32)
    c = jnp.dot(cnt, cemb_ref[:, :], preferred_element_type=jnp.float32) / 20.0
    ph = (post_ref[:, :] == lax.broadcasted_iota(jnp.int32, (N, 8), 1)
          ).astype(jnp.float32)
    p = jnp.dot(ph, pemb_ref[:, :], preferred_element_type=jnp.float32)
    x_ref[:, :] = jnp.concatenate([wt_ref[:, :], c, p], axis=-1)
    xw_ref[:, :] = jnp.dot(x_ref[:, :], wih_ref[:, :],
                           preferred_element_type=jnp.float32)
    whh = whh_ref[:, :]
    bias = bias_ref[:, :]
    zero = jnp.zeros((BS, CTX), jnp.float32)

    def step(t, carry):
        h, cc = carry
        tf = pl.multiple_of(t * BS, BS)
        tb = pl.multiple_of((LS - 1 - t) * BS, BS)
        xf = xw_ref[pl.ds(tf, BS), 0:512]
        xb = xw_ref[pl.ds(tb, BS), 512:1024]
        xcat = jnp.concatenate([xf, xb], axis=0)
        hf = h[0:BS]
        hb = h[BS:2 * BS]
        xblk = jnp.concatenate([
            jnp.concatenate([hf, zero], axis=1),
            jnp.concatenate([zero, hb], axis=1)], axis=0)
        gates = (xcat + jnp.dot(xblk, whh,
                                preferred_element_type=jnp.float32)) + bias
        ii = jax.nn.sigmoid(gates[:, 0:128])
        ff = jax.nn.sigmoid(gates[:, 128:256])
        gg = jnp.tanh(gates[:, 256:384])
        oo = jax.nn.sigmoid(gates[:, 384:512])
        cn = ff * cc + ii * gg
        hn = oo * jnp.tanh(cn)
        mf = mask_ref[pl.ds(tf, BS), :]
        mb = mask_ref[pl.ds(tb, BS), :]
        mt = jnp.concatenate([mf, mb], axis=0)
        hn = hn * mt + h * (1.0 - mt)
        cn = cn * mt + cc * (1.0 - mt)
        ctx_ref[pl.ds(tf, BS), 0:128] = hn[0:BS]
        ctx_ref[pl.ds(tb, BS), 128:256] = hn[BS:2 * BS]
        return hn, cn

    z64 = jnp.zeros((2 * BS, CTX), jnp.float32)
    lax.fori_loop(0, LS, step, (z64, z64))


def _lstm_call(w_t, char_t, pos_t, mask_t, char_emb, pos_emb8, wih_cat,
               whh_blk, bias_blk):
    return pl.pallas_call(
        _lstm_kernel,
        out_shape=jax.ShapeDtypeStruct((N, HS), jnp.float32),
        scratch_shapes=[
            pltpu.VMEM((N, INP), jnp.float32),
            pltpu.VMEM((N, 1024), jnp.float32),
        ],
    )(w_t, char_t, pos_t, mask_t, char_emb, pos_emb8, wih_cat, whh_blk,
      bias_blk)


_DEC_R = 128
_DEC_NT = N // _DEC_R
_MRG_R = 256


def _decode_kernel(cnt_ref, idx_ref, c0_ref, c1_ref, c2_ref,
                   dw0_ref, db0_ref, t0_ref, v0_ref,
                   dw1_ref, db1_ref, t1_ref, v1_ref,
                   dw2_ref, db2_ref, t2_ref, v2_ref,
                   dw3_ref, db3_ref, t3_ref, v3_ref,
                   xe_ref, ym_ref):
    R = _DEC_R
    gi_dyn = pl.program_id(0)
    t = pl.program_id(1)
    dws = (dw0_ref, dw1_ref, dw2_ref, dw3_ref)
    dbs = (db0_ref, db1_ref, db2_ref, db3_ref)
    ts = (t0_ref, t1_ref, t2_ref, t3_ref)
    vs = (v0_ref, v1_ref, v2_ref, v3_ref)
    idxcol = idx_ref[:, :]
    active = t * R < cnt_ref[gi_dyn]

    for gs in range(4):
        M = GS[gs]

        @pl.when((gi_dyn == gs) & active)
        def _(gs=gs, M=M):
            oh = (idxcol == lax.broadcasted_iota(jnp.int32, (R, N), 1)
                  ).astype(jnp.float32)
            ctxc = (jnp.dot(oh, c0_ref[:, :],
                            preferred_element_type=jnp.float32)
                    + jnp.dot(oh, c1_ref[:, :],
                              preferred_element_type=jnp.float32)) \
                + jnp.dot(oh, c2_ref[:, :], preferred_element_type=jnp.float32)
            z = jnp.dot(ctxc, dws[gs][:, :],
                        preferred_element_type=jnp.float32) + dbs[gs][:, :]
            z = z - jnp.max(z, axis=-1, keepdims=True)
            e = jnp.exp(z)
            probs = e / jnp.sum(e, axis=-1, keepdims=True)
            pspt = jnp.log(probs + 1e-12)
            u = pspt + _gumbel_tile(gs, idxcol, M)
            u2 = u - jnp.max(u, axis=-1, keepdims=True)
            e2 = jnp.exp(u2)
            sft = e2 / jnp.sum(e2, axis=-1, keepdims=True)
            mx = jnp.max(sft, axis=-1, keepdims=True)
            iot = lax.broadcasted_iota(jnp.int32, (R, M), 1)
            packed = iot * _PACK + vs[gs][:, :]
            ym_ref[:, :] = jnp.min(
                jnp.where(sft == mx, packed, jnp.int32(2 ** 31 - 1)),
                axis=-1, keepdims=True) & (_PACK - 1)
            xe_ref[:, :] = jnp.dot(sft, ts[gs][:, :],
                                   preferred_element_type=jnp.float32)


def _decode_call(ctx_splits, counts, idx_flat, dec_Ws, dec_bs, tgts,
                 words_all):
    R = _DEC_R
    rows = lambda gi, t, cnt: (gi * _DEC_NT + t, 0)
    full2 = lambda gi, t, cnt: (0, 0)
    in_specs = [
        pl.BlockSpec((R, 1), rows),
        pl.BlockSpec((N, HS), full2),
        pl.BlockSpec((N, HS), full2),
        pl.BlockSpec((N, HS), full2),
    ]
    args = [idx_flat, *ctx_splits]
    for gi in range(4):
        M = GS[gi]
        in_specs += [
            pl.BlockSpec((HS, M), full2),
            pl.BlockSpec((1, M), full2),
            pl.BlockSpec((M, WDIM), full2),
            pl.BlockSpec((1, M), full2),
        ]
        args += [dec_Ws[gi], dec_bs[gi].reshape(1, M), tgts[gi],
                 words_all[gi].astype(jnp.int32).reshape(1, M)]
    grid_spec = pltpu.PrefetchScalarGridSpec(
        num_scalar_prefetch=1,
        grid=(4, _DEC_NT),
        in_specs=in_specs,
        out_specs=[pl.BlockSpec((R, WDIM), rows), pl.BlockSpec((R, 1), rows)],
    )
    return pl.pallas_call(
        _decode_kernel,
        grid_spec=grid_spec,
        out_shape=[jax.ShapeDtypeStruct((4 * N, WDIM), jnp.float32),
                   jax.ShapeDtypeStruct((4 * N, 1), jnp.int32)],
    )(counts, *args)


def _merge_kernel(wb_ref, pos_ref, word_ref, idx4_ref, ym4_ref, scat_ref,
                  emb_ref, ow_ref, msk_ref, nn_ref):
    R = _MRG_R
    pos = pos_ref[:, :]
    ow = word_ref[:, :]
    rglob = pl.program_id(0) * R + lax.broadcasted_iota(jnp.int32, (R, 1), 0)
    anyhit = (pos >= 1) & (pos <= 4)
    emb_ref[:, :] = jnp.where(anyhit, scat_ref[:, :], wb_ref[:, :])
    for gi in range(4):
        pid = POS_IDS[gi]
        hit = pos == pid
        oh = (idx4_ref[gi:gi + 1, :] == rglob) & hit
        sel = jnp.where(oh, jnp.broadcast_to(ym4_ref[gi:gi + 1, :], (R, N)),
                        jnp.int32(2 ** 31 - 1))
        owg = jnp.min(sel, axis=-1, keepdims=True)
        ow = jnp.where(hit, owg, ow)
    ow_ref[:, :] = ow
    msk_ref[:, :] = anyhit.astype(jnp.float32)
    nn_ref[:, :] = ((pos == 1) | (pos == 2)).astype(jnp.int32)


def _charext_kernel(ow_ref, g_ref, oc_ref):
    off = ow_ref[:, :] & 3
    g = g_ref[:, :]
    oc = g[:, 0:MAXC]
    for o in range(1, 4):
        oc = jnp.where(off == o, g[:, o * 32:o * 32 + MAXC], oc)
    oc_ref[:, :] = oc


def _charext_call(ow_col, gath):
    R = _MRG_R
    rows = lambda i: (i, 0)
    return pl.pallas_call(
        _charext_kernel,
        grid=(N // R,),
        in_specs=[pl.BlockSpec((R, 1), rows), pl.BlockSpec((R, 128), rows)],
        out_specs=pl.BlockSpec((R, MAXC), rows),
        out_shape=jax.ShapeDtypeStruct((N, MAXC), jnp.int32),
    )(ow_col, gath)


def _merge_call(wb, pos_col, word_col, idx4, ym4, emb_scat):
    R = _MRG_R
    rows = lambda i: (i, 0)
    full2 = lambda i: (0, 0)
    return pl.pallas_call(
        _merge_kernel,
        grid=(N // R,),
        in_specs=[
            pl.BlockSpec((R, WDIM), rows),
            pl.BlockSpec((R, 1), rows),
            pl.BlockSpec((R, 1), rows),
            pl.BlockSpec((4, N), full2),
            pl.BlockSpec((4, N), full2),
            pl.BlockSpec((R, WDIM), rows),
        ],
        out_specs=[pl.BlockSpec((R, WDIM), rows), pl.BlockSpec((R, 1), rows),
                   pl.BlockSpec((R, 1), rows), pl.BlockSpec((R, 1), rows)],
        out_shape=[jax.ShapeDtypeStruct((N, WDIM), jnp.float32),
                   jax.ShapeDtypeStruct((N, 1), jnp.int32),
                   jax.ShapeDtypeStruct((N, 1), jnp.float32),
                   jax.ShapeDtypeStruct((N, 1), jnp.int32)],
    )(wb, pos_col, word_col, idx4, ym4, emb_scat)


def kernel(inp_word, inp_char, inp_pos, masks, word_emb_weight, char_emb,
           pos_emb, W_ih_f, W_hh_f, b_f, W_ih_b, W_hh_b, b_b,
           dec_W_0, dec_b_0, dec_W_1, dec_b_1, dec_W_2, dec_b_2,
           dec_W_3, dec_b_3, words_0, words_1, words_2, words_3, lut):
    i32 = jnp.int32
    word_b = inp_word.reshape(N).astype(i32)
    word_tm = inp_word.T.reshape(N).astype(i32)
    words_all = (words_0, words_1, words_2, words_3)
    pad = jnp.zeros((76,), i32)
    idx_all = jnp.concatenate(
        [word_tm] + [w.astype(i32) for w in words_all] + [pad])
    rows = _sc_gather_rows(word_emb_weight, idx_all)
    w_t = rows[:N]
    off = N
    tgts = []
    for g in GS:
        tgts.append(rows[off:off + g])
        off += g

    char_t = inp_char.transpose(1, 0, 2).reshape(N, MAXC).astype(i32)
    pos_t = inp_pos.T.reshape(N, 1).astype(i32)
    mask_t = masks.T.reshape(N, 1).astype(jnp.float32)
    wih_cat = jnp.concatenate([W_ih_f.T, W_ih_b.T], axis=1)
    whh_blk = jnp.concatenate([W_hh_f.T, W_hh_b.T], axis=0)
    bias_blk = jnp.concatenate([
        jnp.broadcast_to(b_f, (BS, 4 * CTX)),
        jnp.broadcast_to(b_b, (BS, 4 * CTX))], axis=0)

    ctx_t = _lstm_call(w_t, char_t, pos_t, mask_t, char_emb, pos_emb[:8],
                       wih_cat, whh_blk, bias_blk)
    ctx_b = ctx_t.reshape(LS, BS, HS).transpose(1, 0, 2).reshape(N, HS)
    wb = w_t.reshape(LS, BS, WDIM).transpose(1, 0, 2).reshape(N, WDIM)

    pos_b = inp_pos.reshape(N).astype(i32)
    masks_g = [pos_b == pid for pid in POS_IDS]
    counts = jnp.stack([jnp.sum(m).astype(i32) for m in masks_g])
    idx4 = jnp.stack([jnp.argsort(jnp.logical_not(m)) for m in masks_g]
                     ).astype(i32)
    slot = jnp.zeros((N,), i32)
    for gi in range(4):
        inv_gi = jnp.cumsum(masks_g[gi].astype(i32)) - 1
        slot = jnp.where(masks_g[gi], gi * N + inv_gi, slot)
    c0 = (ctx_b.astype(jnp.bfloat16)).astype(jnp.float32)
    r1 = ctx_b - c0
    c1 = (r1.astype(jnp.bfloat16)).astype(jnp.float32)
    c2 = r1 - c1
    dec_Ws = (dec_W_0, dec_W_1, dec_W_2, dec_W_3)
    dec_bs = (dec_b_0, dec_b_1, dec_b_2, dec_b_3)
    xe_c, ym_c = _decode_call((c0, c1, c2), counts, idx4.reshape(4 * N, 1),
                              dec_Ws, dec_bs, tgts, words_all)
    emb_sel = _sc_gather_rows(xe_c, slot)
    emb_flat, ow, msk_flat, nn_flat = _merge_call(
        wb, inp_pos.reshape(N, 1).astype(i32), word_b.reshape(N, 1),
        idx4, ym_c.reshape(4, N), emb_sel)

    lut32 = jnp.pad(lut.astype(i32), ((0, 0), (0, 32 - MAXC))
                    ).reshape(25000, 128)
    gath = _sc_gather_rows(lut32, ow.reshape(N) >> 2)
    oc = _charext_call(ow, gath)

    obf_word = ow.reshape(BS, LS).astype(inp_word.dtype)
    obf_char = oc.reshape(BS, LS, MAXC).astype(lut.dtype)
    obf_word_emb = emb_flat.reshape(BS, LS, WDIM)
    msk = msk_flat.reshape(BS, LS)
    nn_msk = nn_flat.reshape(BS, LS).astype(bool)
    return (obf_word, obf_char, inp_pos, obf_word_emb, msk, nn_msk)


# speedup vs baseline: 1.0224x; 1.0224x over previous
import functools

import jax
import jax.numpy as jnp
import numpy as np
from jax import lax
from jax.experimental import pallas as pl
from jax.experimental.pallas import tpu as pltpu
from jax.experimental.pallas import tpu_sc as plsc

BS, LS = 32, 64
N = BS * LS
WDIM, CDIM, PDIM = 128, 32, 48
MAXC = 20
INP = WDIM + CDIM + PDIM
CTX = 128
HS = 2 * CTX
GS = (2000, 1500, 1000, 800)
POS_IDS = (1, 2, 3, 4)
NW = 32


def _sc_gather_rows(table, idx):
    B = idx.shape[0]
    D = table.shape[1]
    bpw = B // NW
    mesh = plsc.VectorSubcoreMesh(core_axis_name="c", subcore_axis_name="s")

    @functools.partial(
        pl.kernel,
        out_type=jax.ShapeDtypeStruct((B, D), table.dtype),
        mesh=mesh,
        scratch_types=[
            pltpu.VMEM((bpw,), jnp.int32),
            pltpu.VMEM((bpw, D), table.dtype),
            pltpu.SemaphoreType.DMA,
        ],
    )
    def k(table_hbm, idx_hbm, out_hbm, idx_v, rows_v, sem):
        wid = lax.axis_index("s") * 2 + lax.axis_index("c")
        base = wid * bpw
        pltpu.sync_copy(idx_hbm.at[pl.ds(base, bpw)], idx_v)
        pltpu.async_copy(table_hbm.at[idx_v], rows_v, sem).wait()
        pltpu.sync_copy(rows_v, out_hbm.at[pl.ds(base, bpw)])

    return k(table, idx)


_PACK = 1 << 17


def _np_threefry(k1, k2, x0, x1):
    k1 = np.uint32(k1)
    k2 = np.uint32(k2)
    x0 = np.uint32(x0)
    x1 = np.uint32(x1)
    ks = (k1, k2, np.uint32(k1 ^ k2 ^ np.uint32(0x1BD11BDA)))
    r1 = (13, 15, 26, 6)
    r2 = (17, 29, 16, 24)

    def rol(v, d):
        return np.uint32((np.uint32(v << np.uint32(d)))
                         | np.uint32(v >> np.uint32(32 - d)))

    def rounds(a, b, rots):
        for r in rots:
            a = np.uint32(a + b)
            b = rol(b, r)
            b = np.uint32(a ^ b)
        return a, b

    x0 = np.uint32(x0 + ks[0])
    x1 = np.uint32(x1 + ks[1])
    for i, rots in enumerate((r1, r2, r1, r2, r1)):
        x0, x1 = rounds(x0, x1, rots)
        x0 = np.uint32(x0 + ks[(i + 1) % 3])
        x1 = np.uint32(x1 + ks[(i + 2) % 3] + np.uint32(i + 1))
    return x0, x1


with np.errstate(over="ignore"):
    _GKEYS = tuple(_np_threefry(0, 42, 0, gi) for gi in range(4))
_TINY = np.float32(np.finfo(np.float32).tiny)


def _gumbel_tile(gi, rowids, M):
    rows = rowids.shape[0]
    k1, k2 = _GKEYS[gi]
    ks0 = jnp.uint32(k1)
    ks1 = jnp.uint32(k2)
    ks2 = jnp.uint32(np.uint32(k1) ^ np.uint32(k2) ^ np.uint32(0x1BD11BDA))
    j = lax.broadcasted_iota(jnp.int32, (rows, M), 1)
    n = (rowids * M + j).astype(jnp.uint32)

    def rol(v, d):
        return (v << jnp.uint32(d)) | (v >> jnp.uint32(32 - d))

    def rounds(a, b, rots):
        for rr in rots:
            a = a + b
            b = rol(b, rr)
            b = a ^ b
        return a, b

    x0 = jnp.full((rows, M), ks0, jnp.uint32)
    x1 = n + ks1
    ksl = (ks0, ks1, ks2)
    r1 = (13, 15, 26, 6)
    r2 = (17, 29, 16, 24)
    for i, rots in enumerate((r1, r2, r1, r2, r1)):
        x0, x1 = rounds(x0, x1, rots)
        x0 = x0 + ksl[(i + 1) % 3]
        x1 = x1 + (ksl[(i + 2) % 3] + jnp.uint32(i + 1))
    bits = x0 ^ x1
    fb = (bits >> jnp.uint32(9)) | jnp.uint32(0x3F800000)
    f = lax.bitcast_convert_type(fb, jnp.float32) - 1.0
    u = jnp.maximum(_TINY, f * np.float32(1.0 - _TINY) + _TINY)
    return -jnp.log(-jnp.log(u))


def _lstm_kernel(wt_ref, char_ref, post_ref, mask_ref, cemb_ref, pemb_ref,
                 wih_ref, whh_ref, bias_ref, ctx_ref, x_ref, xw_ref):
    cnt = jnp.zeros((N, 128), jnp.float32)
    for j in range(MAXC):
        col = char_ref[:, j][:, None]
        cnt = cnt + (col == lax.broadcasted_iota(jnp.int32, (N, 128), 1)
                     ).astype(jnp.float32)
    c = jnp.dot(cnt, cemb_ref[:, :], preferred_element_type=jnp.float32) / 20.0
    ph = (post_ref[:, :] == lax.broadcasted_iota(jnp.int32, (N, 8), 1)
          ).astype(jnp.float32)
    p = jnp.dot(ph, pemb_ref[:, :], preferred_element_type=jnp.float32)
    x_ref[:, :] = jnp.concatenate([wt_ref[:, :], c, p], axis=-1)
    xw_ref[:, :] = jnp.dot(x_ref[:, :], wih_ref[:, :],
                           preferred_element_type=jnp.float32)
    whh = whh_ref[:, :]
    bias = bias_ref[:, :]
    zero = jnp.zeros((BS, CTX), jnp.float32)

    def step(t, carry):
        h, cc = carry
        tf = pl.multiple_of(t * BS, BS)
        tb = pl.multiple_of((LS - 1 - t) * BS, BS)
        xf = xw_ref[pl.ds(tf, BS), 0:512]
        xb = xw_ref[pl.ds(tb, BS), 512:1024]
        xcat = jnp.concatenate([xf, xb], axis=0)
        hf = h[0:BS]
        hb = h[BS:2 * BS]
        xblk = jnp.concatenate([
            jnp.concatenate([hf, zero], axis=1),
            jnp.concatenate([zero, hb], axis=1)], axis=0)
        gates = (xcat + jnp.dot(xblk, whh,
                                preferred_element_type=jnp.float32)) + bias
        ii = jax.nn.sigmoid(gates[:, 0:128])
        ff = jax.nn.sigmoid(gates[:, 128:256])
        gg = jnp.tanh(gates[:, 256:384])
        oo = jax.nn.sigmoid(gates[:, 384:512])
        cn = ff * cc + ii * gg
        hn = oo * jnp.tanh(cn)
        mf = mask_ref[pl.ds(tf, BS), :]
        mb = mask_ref[pl.ds(tb, BS), :]
        mt = jnp.concatenate([mf, mb], axis=0)
        hn = hn * mt + h * (1.0 - mt)
        cn = cn * mt + cc * (1.0 - mt)
        ctx_ref[pl.ds(tf, BS), 0:128] = hn[0:BS]
        ctx_ref[pl.ds(tb, BS), 128:256] = hn[BS:2 * BS]
        return hn, cn

    z64 = jnp.zeros((2 * BS, CTX), jnp.float32)
    lax.fori_loop(0, LS, step, (z64, z64))


def _lstm_call(w_t, char_t, pos_t, mask_t, char_emb, pos_emb8, wih_cat,
               whh_blk, bias_blk):
    return pl.pallas_call(
        _lstm_kernel,
        out_shape=jax.ShapeDtypeStruct((N, HS), jnp.float32),
        scratch_shapes=[
            pltpu.VMEM((N, INP), jnp.float32),
            pltpu.VMEM((N, 1024), jnp.float32),
        ],
    )(w_t, char_t, pos_t, mask_t, char_emb, pos_emb8, wih_cat, whh_blk,
      bias_blk)


_DEC_R = 256
_DEC_NT = N // _DEC_R
_MRG_R = 256


def _decode_kernel(cnt_ref, idx_ref, c0_ref, c1_ref, c2_ref,
                   dw0_ref, db0_ref, t0_ref, v0_ref,
                   dw1_ref, db1_ref, t1_ref, v1_ref,
                   dw2_ref, db2_ref, t2_ref, v2_ref,
                   dw3_ref, db3_ref, t3_ref, v3_ref,
                   xe_ref, ym_ref):
    R = _DEC_R
    gi_dyn = pl.program_id(0)
    t = pl.program_id(1)
    dws = (dw0_ref, dw1_ref, dw2_ref, dw3_ref)
    dbs = (db0_ref, db1_ref, db2_ref, db3_ref)
    ts = (t0_ref, t1_ref, t2_ref, t3_ref)
    vs = (v0_ref, v1_ref, v2_ref, v3_ref)
    idxcol = idx_ref[:, :]
    active = t * R < cnt_ref[gi_dyn]

    for gs in range(4):
        M = GS[gs]

        @pl.when((gi_dyn == gs) & active)
        def _(gs=gs, M=M):
            oh = (idxcol == lax.broadcasted_iota(jnp.int32, (R, N), 1)
                  ).astype(jnp.float32)
            ctxc = (jnp.dot(oh, c0_ref[:, :],
                            preferred_element_type=jnp.float32)
                    + jnp.dot(oh, c1_ref[:, :],
                              preferred_element_type=jnp.float32)) \
                + jnp.dot(oh, c2_ref[:, :], preferred_element_type=jnp.float32)
            z = jnp.dot(ctxc, dws[gs][:, :],
                        preferred_element_type=jnp.float32) + dbs[gs][:, :]
            z = z - jnp.max(z, axis=-1, keepdims=True)
            e = jnp.exp(z)
            probs = e / jnp.sum(e, axis=-1, keepdims=True)
            pspt = jnp.log(probs + 1e-12)
            u = pspt + _gumbel_tile(gs, idxcol, M)
            u2 = u - jnp.max(u, axis=-1, keepdims=True)
            e2 = jnp.exp(u2)
            sft = e2 / jnp.sum(e2, axis=-1, keepdims=True)
            mx = jnp.max(sft, axis=-1, keepdims=True)
            iot = lax.broadcasted_iota(jnp.int32, (R, M), 1)
            packed = iot * _PACK + vs[gs][:, :]
            ym_ref[:, :] = jnp.min(
                jnp.where(sft == mx, packed, jnp.int32(2 ** 31 - 1)),
                axis=-1, keepdims=True) & (_PACK - 1)
            xe_ref[:, :] = jnp.dot(sft, ts[gs][:, :],
                                   preferred_element_type=jnp.float32)


def _decode_call(ctx_splits, counts, idx_flat, dec_Ws, dec_bs, tgts,
                 words_all):
    R = _DEC_R
    rows = lambda gi, t, cnt: (gi * _DEC_NT + t, 0)
    full2 = lambda gi, t, cnt: (0, 0)
    in_specs = [
        pl.BlockSpec((R, 1), rows),
        pl.BlockSpec((N, HS), full2),
        pl.BlockSpec((N, HS), full2),
        pl.BlockSpec((N, HS), full2),
    ]
    args = [idx_flat, *ctx_splits]
    for gi in range(4):
        M = GS[gi]
        in_specs += [
            pl.BlockSpec((HS, M), full2),
            pl.BlockSpec((1, M), full2),
            pl.BlockSpec((M, WDIM), full2),
            pl.BlockSpec((1, M), full2),
        ]
        args += [dec_Ws[gi], dec_bs[gi].reshape(1, M), tgts[gi],
                 words_all[gi].astype(jnp.int32).reshape(1, M)]
    grid_spec = pltpu.PrefetchScalarGridSpec(
        num_scalar_prefetch=1,
        grid=(4, _DEC_NT),
        in_specs=in_specs,
        out_specs=[pl.BlockSpec((R, WDIM), rows), pl.BlockSpec((R, 1), rows)],
    )
    return pl.pallas_call(
        _decode_kernel,
        grid_spec=grid_spec,
        out_shape=[jax.ShapeDtypeStruct((4 * N, WDIM), jnp.float32),
                   jax.ShapeDtypeStruct((4 * N, 1), jnp.int32)],
    )(counts, *args)


def _merge_kernel(wb_ref, pos_ref, word_ref, idx4_ref, ym4_ref, scat_ref,
                  emb_ref, ow_ref, msk_ref, nn_ref):
    R = _MRG_R
    pos = pos_ref[:, :]
    ow = word_ref[:, :]
    rglob = pl.program_id(0) * R + lax.broadcasted_iota(jnp.int32, (R, 1), 0)
    anyhit = (pos >= 1) & (pos <= 4)
    emb_ref[:, :] = jnp.where(anyhit, scat_ref[:, :], wb_ref[:, :])
    for gi in range(4):
        pid = POS_IDS[gi]
        hit = pos == pid
        oh = (idx4_ref[gi:gi + 1, :] == rglob) & hit
        sel = jnp.where(oh, jnp.broadcast_to(ym4_ref[gi:gi + 1, :], (R, N)),
                        jnp.int32(2 ** 31 - 1))
        owg = jnp.min(sel, axis=-1, keepdims=True)
        ow = jnp.where(hit, owg, ow)
    ow_ref[:, :] = ow
    msk_ref[:, :] = anyhit.astype(jnp.float32)
    nn_ref[:, :] = ((pos == 1) | (pos == 2)).astype(jnp.int32)


def _charext_kernel(ow_ref, g1_ref, g2_ref, oc_ref):
    off = (ow_ref[:, :] * MAXC) & 127
    g = jnp.concatenate([g1_ref[:, :], g2_ref[:, :]], axis=1)
    oc = g[:, 0:MAXC]
    for o in range(4, 128, 4):
        oc = jnp.where(off == o, g[:, o:o + MAXC], oc)
    oc_ref[:, :] = oc


def _charext_call(ow_col, gath2):
    R = _MRG_R
    rows = lambda i: (i, 0)
    half = lambda i: (N // R + i, 0)
    return pl.pallas_call(
        _charext_kernel,
        grid=(N // R,),
        in_specs=[pl.BlockSpec((R, 1), rows),
                  pl.BlockSpec((R, 128), rows),
                  pl.BlockSpec((R, 128), half)],
        out_specs=pl.BlockSpec((R, MAXC), rows),
        out_shape=jax.ShapeDtypeStruct((N, MAXC), jnp.int32),
    )(ow_col, gath2, gath2)


def _merge_call(wb, pos_col, word_col, idx4, ym4, emb_scat):
    R = _MRG_R
    rows = lambda i: (i, 0)
    full2 = lambda i: (0, 0)
    return pl.pallas_call(
        _merge_kernel,
        grid=(N // R,),
        in_specs=[
            pl.BlockSpec((R, WDIM), rows),
            pl.BlockSpec((R, 1), rows),
            pl.BlockSpec((R, 1), rows),
            pl.BlockSpec((4, N), full2),
            pl.BlockSpec((4, N), full2),
            pl.BlockSpec((R, WDIM), rows),
        ],
        out_specs=[pl.BlockSpec((R, WDIM), rows), pl.BlockSpec((R, 1), rows),
                   pl.BlockSpec((R, 1), rows), pl.BlockSpec((R, 1), rows)],
        out_shape=[jax.ShapeDtypeStruct((N, WDIM), jnp.float32),
                   jax.ShapeDtypeStruct((N, 1), jnp.int32),
                   jax.ShapeDtypeStruct((N, 1), jnp.float32),
                   jax.ShapeDtypeStruct((N, 1), jnp.int32)],
    )(wb, pos_col, word_col, idx4, ym4, emb_scat)


def kernel(inp_word, inp_char, inp_pos, masks, word_emb_weight, char_emb,
           pos_emb, W_ih_f, W_hh_f, b_f, W_ih_b, W_hh_b, b_b,
           dec_W_0, dec_b_0, dec_W_1, dec_b_1, dec_W_2, dec_b_2,
           dec_W_3, dec_b_3, words_0, words_1, words_2, words_3, lut):
    i32 = jnp.int32
    word_b = inp_word.reshape(N).astype(i32)
    word_tm = inp_word.T.reshape(N).astype(i32)
    words_all = (words_0, words_1, words_2, words_3)
    pad = jnp.zeros((76,), i32)
    idx_all = jnp.concatenate(
        [word_tm] + [w.astype(i32) for w in words_all] + [pad])
    rows = _sc_gather_rows(word_emb_weight, idx_all)
    w_t = rows[:N]
    off = N
    tgts = []
    for g in GS:
        tgts.append(rows[off:off + g])
        off += g

    char_t = inp_char.transpose(1, 0, 2).reshape(N, MAXC).astype(i32)
    pos_t = inp_pos.T.reshape(N, 1).astype(i32)
    mask_t = masks.T.reshape(N, 1).astype(jnp.float32)
    wih_cat = jnp.concatenate([W_ih_f.T, W_ih_b.T], axis=1)
    whh_blk = jnp.concatenate([W_hh_f.T, W_hh_b.T], axis=0)
    bias_blk = jnp.concatenate([
        jnp.broadcast_to(b_f, (BS, 4 * CTX)),
        jnp.broadcast_to(b_b, (BS, 4 * CTX))], axis=0)

    ctx_t = _lstm_call(w_t, char_t, pos_t, mask_t, char_emb, pos_emb[:8],
                       wih_cat, whh_blk, bias_blk)
    ctx_b = ctx_t.reshape(LS, BS, HS).transpose(1, 0, 2).reshape(N, HS)
    wb = w_t.reshape(LS, BS, WDIM).transpose(1, 0, 2).reshape(N, WDIM)

    pos_b = inp_pos.reshape(N).astype(i32)
    masks_g = [pos_b == pid for pid in POS_IDS]
    counts = jnp.stack([jnp.sum(m).astype(i32) for m in masks_g])
    idx4 = jnp.stack([jnp.argsort(jnp.logical_not(m)) for m in masks_g]
                     ).astype(i32)
    slot = jnp.zeros((N,), i32)
    for gi in range(4):
        inv_gi = jnp.cumsum(masks_g[gi].astype(i32)) - 1
        slot = jnp.where(masks_g[gi], gi * N + inv_gi, slot)
    c0 = (ctx_b.astype(jnp.bfloat16)).astype(jnp.float32)
    r1 = ctx_b - c0
    c1 = (r1.astype(jnp.bfloat16)).astype(jnp.float32)
    c2 = r1 - c1
    dec_Ws = (dec_W_0, dec_W_1, dec_W_2, dec_W_3)
    dec_bs = (dec_b_0, dec_b_1, dec_b_2, dec_b_3)
    xe_c, ym_c = _decode_call((c0, c1, c2), counts, idx4.reshape(4 * N, 1),
                              dec_Ws, dec_bs, tgts, words_all)
    emb_sel = _sc_gather_rows(xe_c, slot)
    emb_flat, ow, msk_flat, nn_flat = _merge_call(
        wb, inp_pos.reshape(N, 1).astype(i32), word_b.reshape(N, 1),
        idx4, ym_c.reshape(4, N), emb_sel)

    lutf = lut.astype(i32).reshape(15625, 128)
    owf = ow.reshape(N)
    r1 = (owf * MAXC) >> 7
    r2 = jnp.minimum(r1 + 1, 15624)
    gath2 = _sc_gather_rows(lutf, jnp.concatenate([r1, r2]))
    oc = _charext_call(ow, gath2)

    obf_word = ow.reshape(BS, LS).astype(inp_word.dtype)
    obf_char = oc.reshape(BS, LS, MAXC).astype(lut.dtype)
    obf_word_emb = emb_flat.reshape(BS, LS, WDIM)
    msk = msk_flat.reshape(BS, LS)
    nn_msk = nn_flat.reshape(BS, LS).astype(bool)
    return (obf_word, obf_char, inp_pos, obf_word_emb, msk, nn_msk)


# speedup vs baseline: 1.1068x; 1.0826x over previous
import functools

import jax
import jax.numpy as jnp
import numpy as np
from jax import lax
from jax.experimental import pallas as pl
from jax.experimental.pallas import tpu as pltpu
from jax.experimental.pallas import tpu_sc as plsc

BS, LS = 32, 64
N = BS * LS
WDIM, CDIM, PDIM = 128, 32, 48
MAXC = 20
INP = WDIM + CDIM + PDIM
CTX = 128
HS = 2 * CTX
GS = (2000, 1500, 1000, 800)
POS_IDS = (1, 2, 3, 4)
NW = 32


def _sc_gather_rows(table, idx):
    B = idx.shape[0]
    D = table.shape[1]
    bpw = B // NW
    mesh = plsc.VectorSubcoreMesh(core_axis_name="c", subcore_axis_name="s")

    @functools.partial(
        pl.kernel,
        out_type=jax.ShapeDtypeStruct((B, D), table.dtype),
        mesh=mesh,
        scratch_types=[
            pltpu.VMEM((bpw,), jnp.int32),
            pltpu.VMEM((bpw, D), table.dtype),
            pltpu.SemaphoreType.DMA,
        ],
    )
    def k(table_hbm, idx_hbm, out_hbm, idx_v, rows_v, sem):
        wid = lax.axis_index("s") * 2 + lax.axis_index("c")
        base = wid * bpw
        pltpu.sync_copy(idx_hbm.at[pl.ds(base, bpw)], idx_v)
        pltpu.async_copy(table_hbm.at[idx_v], rows_v, sem).wait()
        pltpu.sync_copy(rows_v, out_hbm.at[pl.ds(base, bpw)])

    return k(table, idx)


_PACK = 1 << 17


def _np_threefry(k1, k2, x0, x1):
    k1 = np.uint32(k1)
    k2 = np.uint32(k2)
    x0 = np.uint32(x0)
    x1 = np.uint32(x1)
    ks = (k1, k2, np.uint32(k1 ^ k2 ^ np.uint32(0x1BD11BDA)))
    r1 = (13, 15, 26, 6)
    r2 = (17, 29, 16, 24)

    def rol(v, d):
        return np.uint32((np.uint32(v << np.uint32(d)))
                         | np.uint32(v >> np.uint32(32 - d)))

    def rounds(a, b, rots):
        for r in rots:
            a = np.uint32(a + b)
            b = rol(b, r)
            b = np.uint32(a ^ b)
        return a, b

    x0 = np.uint32(x0 + ks[0])
    x1 = np.uint32(x1 + ks[1])
    for i, rots in enumerate((r1, r2, r1, r2, r1)):
        x0, x1 = rounds(x0, x1, rots)
        x0 = np.uint32(x0 + ks[(i + 1) % 3])
        x1 = np.uint32(x1 + ks[(i + 2) % 3] + np.uint32(i + 1))
    return x0, x1


with np.errstate(over="ignore"):
    _GKEYS = tuple(_np_threefry(0, 42, 0, gi) for gi in range(4))
_TINY = np.float32(np.finfo(np.float32).tiny)


def _gumbel_tile(gi, rowids, M):
    rows = rowids.shape[0]
    k1, k2 = _GKEYS[gi]
    ks0 = jnp.uint32(k1)
    ks1 = jnp.uint32(k2)
    ks2 = jnp.uint32(np.uint32(k1) ^ np.uint32(k2) ^ np.uint32(0x1BD11BDA))
    j = lax.broadcasted_iota(jnp.int32, (rows, M), 1)
    n = (rowids * M + j).astype(jnp.uint32)

    def rol(v, d):
        return (v << jnp.uint32(d)) | (v >> jnp.uint32(32 - d))

    def rounds(a, b, rots):
        for rr in rots:
            a = a + b
            b = rol(b, rr)
            b = a ^ b
        return a, b

    x0 = jnp.full((rows, M), ks0, jnp.uint32)
    x1 = n + ks1
    ksl = (ks0, ks1, ks2)
    r1 = (13, 15, 26, 6)
    r2 = (17, 29, 16, 24)
    for i, rots in enumerate((r1, r2, r1, r2, r1)):
        x0, x1 = rounds(x0, x1, rots)
        x0 = x0 + ksl[(i + 1) % 3]
        x1 = x1 + (ksl[(i + 2) % 3] + jnp.uint32(i + 1))
    bits = x0 ^ x1
    fb = (bits >> jnp.uint32(9)) | jnp.uint32(0x3F800000)
    f = lax.bitcast_convert_type(fb, jnp.float32) - 1.0
    u = jnp.maximum(_TINY, f * np.float32(1.0 - _TINY) + _TINY)
    return -jnp.log(-jnp.log(u))


def _lstm_kernel(wt_ref, char_ref, post_ref, mask_ref, cemb_ref, pemb_ref,
                 wih_ref, whh_ref, bias_ref, ctx_ref, x_ref, xw_ref):
    cnt = jnp.zeros((N, 128), jnp.float32)
    for j in range(MAXC):
        col = char_ref[:, j][:, None]
        cnt = cnt + (col == lax.broadcasted_iota(jnp.int32, (N, 128), 1)
                     ).astype(jnp.float32)
    c = jnp.dot(cnt, cemb_ref[:, :], preferred_element_type=jnp.float32) / 20.0
    ph = (post_ref[:, :] == lax.broadcasted_iota(jnp.int32, (N, 8), 1)
          ).astype(jnp.float32)
    p = jnp.dot(ph, pemb_ref[:, :], preferred_element_type=jnp.float32)
    x_ref[:, :] = jnp.concatenate([wt_ref[:, :], c, p], axis=-1)
    xw_ref[:, :] = jnp.dot(x_ref[:, :], wih_ref[:, :],
                           preferred_element_type=jnp.float32)
    whh = whh_ref[:, :]
    bias = bias_ref[:, :]
    zero = jnp.zeros((BS, CTX), jnp.float32)

    def step(t, carry):
        h, cc = carry
        tf = pl.multiple_of(t * BS, BS)
        tb = pl.multiple_of((LS - 1 - t) * BS, BS)
        xf = xw_ref[pl.ds(tf, BS), 0:512]
        xb = xw_ref[pl.ds(tb, BS), 512:1024]
        xcat = jnp.concatenate([xf, xb], axis=0)
        hf = h[0:BS]
        hb = h[BS:2 * BS]
        xblk = jnp.concatenate([
            jnp.concatenate([hf, zero], axis=1),
            jnp.concatenate([zero, hb], axis=1)], axis=0)
        gates = (xcat + jnp.dot(xblk, whh,
                                preferred_element_type=jnp.float32)) + bias
        ii = jax.nn.sigmoid(gates[:, 0:128])
        ff = jax.nn.sigmoid(gates[:, 128:256])
        gg = jnp.tanh(gates[:, 256:384])
        oo = jax.nn.sigmoid(gates[:, 384:512])
        cn = ff * cc + ii * gg
        hn = oo * jnp.tanh(cn)
        mf = mask_ref[pl.ds(tf, BS), :]
        mb = mask_ref[pl.ds(tb, BS), :]
        mt = jnp.concatenate([mf, mb], axis=0)
        hn = hn * mt + h * (1.0 - mt)
        cn = cn * mt + cc * (1.0 - mt)
        ctx_ref[pl.ds(tf, BS), 0:128] = hn[0:BS]
        ctx_ref[pl.ds(tb, BS), 128:256] = hn[BS:2 * BS]
        return hn, cn

    z64 = jnp.zeros((2 * BS, CTX), jnp.float32)
    lax.fori_loop(0, LS, step, (z64, z64))


def _lstm_call(w_t, char_t, pos_t, mask_t, char_emb, pos_emb8, wih_cat,
               whh_blk, bias_blk):
    return pl.pallas_call(
        _lstm_kernel,
        out_shape=jax.ShapeDtypeStruct((N, HS), jnp.float32),
        scratch_shapes=[
            pltpu.VMEM((N, INP), jnp.float32),
            pltpu.VMEM((N, 1024), jnp.float32),
        ],
    )(w_t, char_t, pos_t, mask_t, char_emb, pos_emb8, wih_cat, whh_blk,
      bias_blk)


_DEC_R = 128
_DEC_NT = N // _DEC_R
_MRG_R = 256


def _decode_kernel(cnt_ref, idx_ref, c0_ref, c1_ref, c2_ref,
                   dw0_ref, db0_ref, t0_ref, v0_ref,
                   dw1_ref, db1_ref, t1_ref, v1_ref,
                   dw2_ref, db2_ref, t2_ref, v2_ref,
                   dw3_ref, db3_ref, t3_ref, v3_ref,
                   xe_ref, ym_ref):
    R = _DEC_R
    gi_dyn = pl.program_id(0)
    t = pl.program_id(1)
    dws = (dw0_ref, dw1_ref, dw2_ref, dw3_ref)
    dbs = (db0_ref, db1_ref, db2_ref, db3_ref)
    ts = (t0_ref, t1_ref, t2_ref, t3_ref)
    vs = (v0_ref, v1_ref, v2_ref, v3_ref)
    idxcol = idx_ref[:, :]
    active = t * R < cnt_ref[gi_dyn]

    for gs in range(4):
        M = GS[gs]

        @pl.when((gi_dyn == gs) & active)
        def _(gs=gs, M=M):
            oh = (idxcol == lax.broadcasted_iota(jnp.int32, (R, N), 1)
                  ).astype(jnp.float32)
            ctxc = (jnp.dot(oh, c0_ref[:, :],
                            preferred_element_type=jnp.float32)
                    + jnp.dot(oh, c1_ref[:, :],
                              preferred_element_type=jnp.float32)) \
                + jnp.dot(oh, c2_ref[:, :], preferred_element_type=jnp.float32)
            z = jnp.dot(ctxc, dws[gs][:, :],
                        preferred_element_type=jnp.float32) + dbs[gs][:, :]
            z = z - jnp.max(z, axis=-1, keepdims=True)
            e = jnp.exp(z)
            probs = e / jnp.sum(e, axis=-1, keepdims=True)
            pspt = jnp.log(probs + 1e-12)
            u = pspt + _gumbel_tile(gs, idxcol, M)
            u2 = u - jnp.max(u, axis=-1, keepdims=True)
            e2 = jnp.exp(u2)
            sft = e2 / jnp.sum(e2, axis=-1, keepdims=True)
            mx = jnp.max(sft, axis=-1, keepdims=True)
            iot = lax.broadcasted_iota(jnp.int32, (R, M), 1)
            packed = iot * _PACK + vs[gs][:, :]
            ym_ref[:, :] = jnp.min(
                jnp.where(sft == mx, packed, jnp.int32(2 ** 31 - 1)),
                axis=-1, keepdims=True) & (_PACK - 1)
            xe_ref[:, :] = jnp.dot(sft, ts[gs][:, :],
                                   preferred_element_type=jnp.float32)


def _decode_call(ctx_splits, counts, idx_flat, dec_Ws, dec_bs, tgts,
                 words_all):
    R = _DEC_R
    rows = lambda gi, t, cnt: (gi * _DEC_NT + t, 0)
    full2 = lambda gi, t, cnt: (0, 0)
    in_specs = [
        pl.BlockSpec((R, 1), rows),
        pl.BlockSpec((N, HS), full2),
        pl.BlockSpec((N, HS), full2),
        pl.BlockSpec((N, HS), full2),
    ]
    args = [idx_flat, *ctx_splits]
    for gi in range(4):
        M = GS[gi]
        in_specs += [
            pl.BlockSpec((HS, M), full2),
            pl.BlockSpec((1, M), full2),
            pl.BlockSpec((M, WDIM), full2),
            pl.BlockSpec((1, M), full2),
        ]
        args += [dec_Ws[gi], dec_bs[gi].reshape(1, M), tgts[gi],
                 words_all[gi].astype(jnp.int32).reshape(1, M)]
    grid_spec = pltpu.PrefetchScalarGridSpec(
        num_scalar_prefetch=1,
        grid=(4, _DEC_NT),
        in_specs=in_specs,
        out_specs=[pl.BlockSpec((R, WDIM), rows), pl.BlockSpec((R, 1), rows)],
    )
    return pl.pallas_call(
        _decode_kernel,
        grid_spec=grid_spec,
        out_shape=[jax.ShapeDtypeStruct((4 * N, WDIM), jnp.float32),
                   jax.ShapeDtypeStruct((4 * N, 1), jnp.int32)],
    )(counts, *args)


def _merge_kernel(wb_ref, pos_ref, word_ref, idx4_ref, ym4_ref, scat_ref,
                  emb_ref, ow_ref, msk_ref, nn_ref):
    R = _MRG_R
    pos = pos_ref[:, :]
    ow = word_ref[:, :]
    rglob = pl.program_id(0) * R + lax.broadcasted_iota(jnp.int32, (R, 1), 0)
    anyhit = (pos >= 1) & (pos <= 4)
    emb_ref[:, :] = jnp.where(anyhit, scat_ref[:, :], wb_ref[:, :])
    for gi in range(4):
        pid = POS_IDS[gi]
        hit = pos == pid
        oh = (idx4_ref[gi:gi + 1, :] == rglob) & hit
        sel = jnp.where(oh, jnp.broadcast_to(ym4_ref[gi:gi + 1, :], (R, N)),
                        jnp.int32(2 ** 31 - 1))
        owg = jnp.min(sel, axis=-1, keepdims=True)
        ow = jnp.where(hit, owg, ow)
    ow_ref[:, :] = ow
    msk_ref[:, :] = anyhit.astype(jnp.float32)
    nn_ref[:, :] = ((pos == 1) | (pos == 2)).astype(jnp.int32)


def _charext_kernel(ow_ref, g1_ref, g2_ref, oc_ref):
    off = (ow_ref[:, :] * MAXC) & 127
    g = jnp.concatenate([g1_ref[:, :], g2_ref[:, :]], axis=1)
    for b in (64, 32, 16, 8, 4):
        w = g.shape[1]
        g = jnp.where((off & b) != 0, g[:, b:], g[:, :w - b])
    oc_ref[:, :] = g[:, 0:MAXC]


def _charext_call(ow_col, gath2):
    R = _MRG_R
    rows = lambda i: (i, 0)
    half = lambda i: (N // R + i, 0)
    return pl.pallas_call(
        _charext_kernel,
        grid=(N // R,),
        in_specs=[pl.BlockSpec((R, 1), rows),
                  pl.BlockSpec((R, 128), rows),
                  pl.BlockSpec((R, 128), half)],
        out_specs=pl.BlockSpec((R, MAXC), rows),
        out_shape=jax.ShapeDtypeStruct((N, MAXC), jnp.int32),
    )(ow_col, gath2, gath2)


def _merge_call(wb, pos_col, word_col, idx4, ym4, emb_scat):
    R = _MRG_R
    rows = lambda i: (i, 0)
    full2 = lambda i: (0, 0)
    return pl.pallas_call(
        _merge_kernel,
        grid=(N // R,),
        in_specs=[
            pl.BlockSpec((R, WDIM), rows),
            pl.BlockSpec((R, 1), rows),
            pl.BlockSpec((R, 1), rows),
            pl.BlockSpec((4, N), full2),
            pl.BlockSpec((4, N), full2),
            pl.BlockSpec((R, WDIM), rows),
        ],
        out_specs=[pl.BlockSpec((R, WDIM), rows), pl.BlockSpec((R, 1), rows),
                   pl.BlockSpec((R, 1), rows), pl.BlockSpec((R, 1), rows)],
        out_shape=[jax.ShapeDtypeStruct((N, WDIM), jnp.float32),
                   jax.ShapeDtypeStruct((N, 1), jnp.int32),
                   jax.ShapeDtypeStruct((N, 1), jnp.float32),
                   jax.ShapeDtypeStruct((N, 1), jnp.int32)],
    )(wb, pos_col, word_col, idx4, ym4, emb_scat)


def kernel(inp_word, inp_char, inp_pos, masks, word_emb_weight, char_emb,
           pos_emb, W_ih_f, W_hh_f, b_f, W_ih_b, W_hh_b, b_b,
           dec_W_0, dec_b_0, dec_W_1, dec_b_1, dec_W_2, dec_b_2,
           dec_W_3, dec_b_3, words_0, words_1, words_2, words_3, lut):
    i32 = jnp.int32
    word_b = inp_word.reshape(N).astype(i32)
    word_tm = inp_word.T.reshape(N).astype(i32)
    words_all = (words_0, words_1, words_2, words_3)
    pad = jnp.zeros((76,), i32)
    idx_all = jnp.concatenate(
        [word_tm] + [w.astype(i32) for w in words_all] + [pad])
    rows = _sc_gather_rows(word_emb_weight, idx_all)
    w_t = rows[:N]
    off = N
    tgts = []
    for g in GS:
        tgts.append(rows[off:off + g])
        off += g

    char_t = inp_char.transpose(1, 0, 2).reshape(N, MAXC).astype(i32)
    pos_t = inp_pos.T.reshape(N, 1).astype(i32)
    mask_t = masks.T.reshape(N, 1).astype(jnp.float32)
    wih_cat = jnp.concatenate([W_ih_f.T, W_ih_b.T], axis=1)
    whh_blk = jnp.concatenate([W_hh_f.T, W_hh_b.T], axis=0)
    bias_blk = jnp.concatenate([
        jnp.broadcast_to(b_f, (BS, 4 * CTX)),
        jnp.broadcast_to(b_b, (BS, 4 * CTX))], axis=0)

    ctx_t = _lstm_call(w_t, char_t, pos_t, mask_t, char_emb, pos_emb[:8],
                       wih_cat, whh_blk, bias_blk)
    ctx_b = ctx_t.reshape(LS, BS, HS).transpose(1, 0, 2).reshape(N, HS)
    wb = w_t.reshape(LS, BS, WDIM).transpose(1, 0, 2).reshape(N, WDIM)

    pos_b = inp_pos.reshape(N).astype(i32)
    masks_g = [pos_b == pid for pid in POS_IDS]
    counts = jnp.stack([jnp.sum(m).astype(i32) for m in masks_g])
    idx4 = jnp.stack([jnp.argsort(jnp.logical_not(m)) for m in masks_g]
                     ).astype(i32)
    slot = jnp.zeros((N,), i32)
    for gi in range(4):
        inv_gi = jnp.cumsum(masks_g[gi].astype(i32)) - 1
        slot = jnp.where(masks_g[gi], gi * N + inv_gi, slot)
    c0 = (ctx_b.astype(jnp.bfloat16)).astype(jnp.float32)
    r1 = ctx_b - c0
    c1 = (r1.astype(jnp.bfloat16)).astype(jnp.float32)
    c2 = r1 - c1
    dec_Ws = (dec_W_0, dec_W_1, dec_W_2, dec_W_3)
    dec_bs = (dec_b_0, dec_b_1, dec_b_2, dec_b_3)
    xe_c, ym_c = _decode_call((c0, c1, c2), counts, idx4.reshape(4 * N, 1),
                              dec_Ws, dec_bs, tgts, words_all)
    emb_sel = _sc_gather_rows(xe_c, slot)
    emb_flat, ow, msk_flat, nn_flat = _merge_call(
        wb, inp_pos.reshape(N, 1).astype(i32), word_b.reshape(N, 1),
        idx4, ym_c.reshape(4, N), emb_sel)

    lutf = lut.astype(i32).reshape(15625, 128)
    owf = ow.reshape(N)
    r1 = (owf * MAXC) >> 7
    r2 = jnp.minimum(r1 + 1, 15624)
    gath2 = _sc_gather_rows(lutf, jnp.concatenate([r1, r2]))
    oc = _charext_call(ow, gath2)

    obf_word = ow.reshape(BS, LS).astype(inp_word.dtype)
    obf_char = oc.reshape(BS, LS, MAXC).astype(lut.dtype)
    obf_word_emb = emb_flat.reshape(BS, LS, WDIM)
    msk = msk_flat.reshape(BS, LS)
    nn_msk = nn_flat.reshape(BS, LS).astype(bool)
    return (obf_word, obf_char, inp_pos, obf_word_emb, msk, nn_msk)
